# baseline (speedup 1.0000x reference)
import functools

import jax
import jax.numpy as jnp
from jax import lax
from jax.experimental import pallas as pl
from jax.experimental.pallas import tpu as pltpu
from jax.experimental.pallas import tpu_sc as plsc

VOCAB = 6147
DIM = 896
BATCH = 1024
SEQ = 200

B = BATCH * SEQ
NC, NS = 2, 16
NW = NC * NS
B_PER_W = B // NW
NGRP = B_PER_W // 16

SLICE = 64
NBKT = (VOCAB + SLICE - 1) // SLICE
NBKT_PAD = 112
SLOT = 256
SLOT_L = 32
POS_MASK = (1 << 18) - 1

_mesh = plsc.VectorSubcoreMesh(core_axis_name="c", subcore_axis_name="s")
_params = pltpu.CompilerParams(needs_layout_passes=False)


@functools.partial(
    pl.kernel,
    mesh=_mesh,
    compiler_params=_params,
    out_type=(
        jax.ShapeDtypeStruct((NW * NBKT_PAD, SLOT), jnp.int32),
        jax.ShapeDtypeStruct((NW, 1, NBKT_PAD), jnp.int32),
    ),
    scratch_types=[
        pltpu.VMEM((1, B_PER_W), jnp.int32),
        pltpu.VMEM((16 * NBKT * SLOT_L,), jnp.int32),
        pltpu.VMEM((NBKT_PAD, SLOT), jnp.int32),
        pltpu.VMEM((16 * NBKT_PAD,), jnp.int32),
        pltpu.VMEM((1, NBKT_PAD), jnp.int32),
        pltpu.VMEM((32,), jnp.int32),
        pltpu.VMEM((32,), jnp.int32),
    ],
)
def _bucketize(idx_hbm, lists_hbm, counts_hbm, idx_v, lane_v, merged_v,
               cntl_v, cnt_v, tmp_c, tmp_o):
    wid = lax.axis_index("s") * NC + lax.axis_index("c")
    base = wid * B_PER_W
    pltpu.sync_copy(idx_hbm.at[wid], idx_v)

    zeros = jnp.zeros((16,), jnp.int32)
    for j in range(16 * NBKT_PAD // 16):
        cntl_v[pl.ds(16 * j, 16)] = zeros
    for j in range(NBKT_PAD // 16):
        cnt_v[0, pl.ds(16 * j, 16)] = zeros
    lane = lax.iota(jnp.int32, 16)

    def body(j, carry):
        v16 = idx_v[0, pl.ds(16 * j, 16)]
        b16 = v16 >> 6
        pk16 = ((v16 & 63) << 18) | (base + 16 * j + lane)
        cidx = lane * NBKT_PAD + b16
        c16 = plsc.load_gather(cntl_v, [cidx])
        plsc.store_scatter(
            lane_v, [(lane * NBKT + b16) * SLOT_L + c16], pk16)
        plsc.store_scatter(cntl_v, [cidx], c16 + 1)
        return carry

    lax.fori_loop(0, NGRP, body, 0)

    def merge(b, carry):
        b16 = jnp.broadcast_to(b, (16,))
        cidx = lane * NBKT_PAD + b
        c16 = plsc.load_gather(cntl_v, [cidx])
        incl = plsc.cumsum(c16)
        off16 = incl - c16
        plsc.store_scatter(cnt_v, [jnp.zeros((16,), jnp.int32), b16], incl,
                           mask=lane == 15)
        tmp_c[pl.ds(0, 16)] = c16
        tmp_o[pl.ds(0, 16)] = off16
        for l in range(16):
            c_l = tmp_c[pl.ds(l, 16)][0]
            o_l = tmp_o[pl.ds(l, 16)][0]
            src_base = (l * NBKT + b) * SLOT_L
            for r in range(SLOT_L // 16):
                e = r * 16 + lane
                msk = e < c_l
                vals = plsc.load_gather(lane_v, [src_base + e], mask=msk)
                plsc.store_scatter(merged_v, [b16, o_l + e], vals, mask=msk)
        return carry

    lax.fori_loop(0, NBKT, merge, 0)
    pltpu.sync_copy(merged_v, lists_hbm.at[pl.ds(wid * NBKT_PAD, NBKT_PAD)])
    pltpu.sync_copy(cnt_v, counts_hbm.at[wid])


@functools.partial(
    pl.kernel,
    mesh=_mesh,
    compiler_params=_params,
    out_type=jax.ShapeDtypeStruct((B, DIM), jnp.float32),
    scratch_types=[
        pltpu.VMEM((SLICE, DIM), jnp.float32),
        pltpu.VMEM((16, DIM), jnp.float32),
        pltpu.VMEM((16, DIM), jnp.float32),
        pltpu.VMEM((2, 16), jnp.int32),
        pltpu.VMEM((NW, SLOT), jnp.int32),
        pltpu.VMEM((NW, 1, NBKT_PAD), jnp.int32),
        pltpu.VMEM((32,), jnp.int32),
        pltpu.VMEM((32,), jnp.int32),
        pltpu.SemaphoreType.DMA,
        pltpu.SemaphoreType.DMA,
    ],
)
def _gather_scatter(lists_hbm, counts_hbm, table_hbm, out_hbm, slice_v,
                    stage0, stage1, pos_v, list_all, cnt_v, vls_v, rowsel_v,
                    sem0, sem1):
    wid = lax.axis_index("s") * NC + lax.axis_index("c")
    pltpu.sync_copy(counts_hbm, cnt_v)
    lane = lax.iota(jnp.int32, 16)
    stages = (stage0, stage1)
    sems = (sem0, sem1)

    def process_slice(s, nrows):
        pltpu.sync_copy(table_hbm.at[pl.ds(s * SLICE, nrows)],
                        slice_v.at[pl.ds(0, nrows)])
        rowsel_v[pl.ds(0, 16)] = lane * NBKT_PAD + s
        rowsel_v[pl.ds(16, 16)] = (lane + 16) * NBKT_PAD + s
        pltpu.async_copy(lists_hbm.at[rowsel_v], list_all, sem0).wait()

        def per_worker(t2, carry):
            n = cnt_v[t2, 0, pl.ds(s, 16)][0]
            t2v = jnp.full((16,), 1, jnp.int32) * t2

            @pl.when(n > 0)
            def _():
                npair = (n + 31) // 32

                def stage_batch(j, buf):
                    ec = jnp.minimum(j * 16 + lane, n - 1)
                    p16 = plsc.load_gather(list_all, [t2v, ec])
                    pos_v[buf, :] = p16 & POS_MASK
                    vls_v[pl.ds(0, 16)] = p16 >> 18

                    def lane_body(l, carry):
                        vl = vls_v[pl.ds(l, 16)][0]
                        for k in range(DIM // 16):
                            stages[buf][l, pl.ds(16 * k, 16)] = (
                                slice_v[vl, pl.ds(16 * k, 16)])
                        return carry

                    lax.fori_loop(0, 16, lane_body, 0)
                    pltpu.async_copy(stages[buf], out_hbm.at[pos_v.at[buf]],
                                     sems[buf])

                stage_batch(0, 0)
                stage_batch(1, 1)

                def pair_body(q, carry):
                    for buf in range(2):
                        pltpu.make_async_copy(
                            stages[buf], out_hbm.at[pos_v.at[buf]],
                            sems[buf]).wait()
                        stage_batch(2 * q + buf, buf)
                    return carry

                lax.fori_loop(1, npair, pair_body, 0)
                for buf in range(2):
                    pltpu.make_async_copy(
                        stages[buf], out_hbm.at[pos_v.at[buf]],
                        sems[buf]).wait()

            return carry

        lax.fori_loop(0, NW, per_worker, 0)

    def per_slice(si, carry):
        process_slice(wid + 32 * si, SLICE)
        return carry

    lax.fori_loop(0, 3, per_slice, 0)

    @pl.when(wid == 0)
    def _():
        process_slice(jnp.int32(NBKT - 1), VOCAB - (NBKT - 1) * SLICE)


def kernel(token_ids, table):
    idx = token_ids.reshape(NW, 1, B_PER_W).astype(jnp.int32)
    lists, counts = _bucketize(idx)
    out = _gather_scatter(lists, counts, table)
    return out.reshape(BATCH, SEQ, DIM)


# speedup vs baseline: 3.7866x; 3.7866x over previous
import functools

import jax
import jax.numpy as jnp
from jax import lax
from jax.experimental import pallas as pl
from jax.experimental.pallas import tpu as pltpu
from jax.experimental.pallas import tpu_sc as plsc

VOCAB = 6147
DIM = 896
BATCH = 1024
SEQ = 200

B = BATCH * SEQ
NC, NS = 2, 16
NW = NC * NS
B_PER_W = B // NW
CHUNK = 32
N_CHUNKS = B_PER_W // CHUNK
NBUF = 3
N_PASSES = N_CHUNKS // NBUF

_mesh = plsc.VectorSubcoreMesh(core_axis_name="c", subcore_axis_name="s")


@functools.partial(
    pl.kernel,
    mesh=_mesh,
    out_type=jax.ShapeDtypeStruct((B, DIM), jnp.float32),
    scratch_types=[
        pltpu.VMEM((N_CHUNKS, CHUNK), jnp.int32),
        *[pltpu.VMEM((CHUNK, DIM), jnp.float32) for _ in range(NBUF)],
        *[pltpu.SemaphoreType.DMA for _ in range(NBUF)],
    ],
)
def _gather_rows(idx_hbm, table_hbm, out_hbm, idx_v, *bufs_and_sems):
    bufs = bufs_and_sems[:NBUF]
    sems = bufs_and_sems[NBUF:]
    wid = lax.axis_index("s") * NC + lax.axis_index("c")
    base = wid * B_PER_W
    pltpu.sync_copy(idx_hbm.at[wid], idx_v)

    for r in range(NBUF):
        pltpu.async_copy(table_hbm.at[idx_v.at[r]], bufs[r], sems[r])

    def body(q, carry):
        c0 = NBUF * q
        for r in range(NBUF):
            c = c0 + r
            pltpu.make_async_copy(
                table_hbm.at[idx_v.at[c]], bufs[r], sems[r]).wait()
            pltpu.sync_copy(bufs[r], out_hbm.at[pl.ds(base + c * CHUNK, CHUNK)])
            nxt = jnp.minimum(c + NBUF, N_CHUNKS - 1)
            pltpu.async_copy(table_hbm.at[idx_v.at[nxt]], bufs[r], sems[r])
        return carry

    lax.fori_loop(0, N_PASSES, body, 0)
    for r in range(NBUF):
        c = N_PASSES * NBUF + r
        pltpu.make_async_copy(
            table_hbm.at[idx_v.at[N_CHUNKS - 1]], bufs[r], sems[r]).wait()
        if c < N_CHUNKS:
            pltpu.sync_copy(bufs[r], out_hbm.at[pl.ds(base + c * CHUNK, CHUNK)])


def kernel(token_ids, table):
    idx = token_ids.reshape(NW, N_CHUNKS, CHUNK).astype(jnp.int32)
    out = _gather_rows(idx, table)
    return out.reshape(BATCH, SEQ, DIM)
